# baseline (speedup 1.0000x reference)
import functools

import jax
import jax.numpy as jnp
from jax import lax
from jax.experimental import pallas as pl
from jax.experimental.pallas import tpu as pltpu
from jax.experimental.pallas import tpu_sc as plsc

N = 10000
E = 320000
IN_DIM = 128
HID = 64
NC = 2
NS = 16
NW = NC * NS
K = 128
NBUF = 2
BPW = 80
EPW = BPW * K
E_PAD = NW * EPW
BPA = 80
BPB = 80
BCH = 40
CFAST = 1
NP = 10240
RPT = NP // NS

_MESH = plsc.VectorSubcoreMesh(core_axis_name="c", subcore_axis_name="s")


@functools.partial(
    pl.kernel,
    mesh=_MESH,
    out_type=jax.ShapeDtypeStruct((NC * NP,), jnp.float32),
    scratch_types=[
        pltpu.VMEM((BPW, K), jnp.int32),
        pltpu.VMEM((K,), jnp.float32),
        pltpu.VMEM_SHARED((NP,), jnp.float32),
    ],
)
def _sc_hist(dst_hbm, ones_hbm, zer_hbm, out_hbm, dst_vm, ones_vm, hist_sh):
    c = lax.axis_index("c")
    s = lax.axis_index("s")
    wid = c * NS + s
    pltpu.sync_copy(dst_hbm.at[wid], dst_vm)
    pltpu.sync_copy(ones_hbm, ones_vm)
    pltpu.sync_copy(zer_hbm, hist_sh.at[pl.ds(s * RPT, RPT)])
    plsc.subcore_barrier()

    def body(j, carry):
        pltpu.sync_copy(ones_vm, hist_sh.at[dst_vm.at[j]], add=True)
        return carry

    lax.fori_loop(0, BPW, body, 0)
    plsc.subcore_barrier()
    pltpu.sync_copy(hist_sh.at[pl.ds(s * RPT, RPT)],
                    out_hbm.at[pl.ds(c * NP + s * RPT, RPT)])


@functools.partial(
    pl.kernel,
    mesh=_MESH,
    out_type=jax.ShapeDtypeStruct((NC * NP, 2 * HID), jnp.float32),
    scratch_types=[
        pltpu.VMEM((BPA, K), jnp.int32),
        pltpu.VMEM((BPA, K), jnp.int32),
        pltpu.VMEM((K, 2 * HID), jnp.float32),
        pltpu.VMEM_SHARED((NP, 2 * HID), jnp.float32),
    ],
)
def _sc_edge1(y_hbm, srca_hbm, dsta_hbm, srcb_hbm, dstb_hbm, zer_hbm, out_hbm,
              src_vm, dst_vm, buf, acc_sh):
    c = lax.axis_index("c")
    s = lax.axis_index("s")
    pltpu.sync_copy(zer_hbm, acc_sh.at[pl.ds(s * RPT, RPT)])
    plsc.subcore_barrier()

    def run(src_hbm, dst_hbm, bpw):
        pltpu.sync_copy(src_hbm.at[s], src_vm.at[pl.ds(0, bpw)])
        pltpu.sync_copy(dst_hbm.at[s], dst_vm.at[pl.ds(0, bpw)])

        def body(j, carry):
            pltpu.sync_copy(y_hbm.at[src_vm.at[j]], buf)
            pltpu.sync_copy(buf, acc_sh.at[dst_vm.at[j]], add=True)
            return carry

        lax.fori_loop(0, bpw, body, 0)

    @pl.when(c == CFAST)
    def _():
        run(srca_hbm, dsta_hbm, BPA)

    @pl.when(c != CFAST)
    def _():
        run(srcb_hbm, dstb_hbm, BPB)

    plsc.subcore_barrier()
    pltpu.sync_copy(acc_sh.at[pl.ds(s * RPT, RPT)],
                    out_hbm.at[pl.ds(c * NP + s * RPT, RPT)])


@functools.partial(
    pl.kernel,
    mesh=_MESH,
    out_type=jax.ShapeDtypeStruct((NC * NP,), jnp.float32),
    scratch_types=[
        pltpu.VMEM((BPW, K), jnp.int32),
        pltpu.VMEM((BPW, K), jnp.int32),
        pltpu.VMEM((K,), jnp.float32),
        pltpu.VMEM_SHARED((N,), jnp.float32),
        pltpu.VMEM_SHARED((NP,), jnp.float32),
    ],
)
def _sc_edge2(z_hbm, src_hbm, dst_hbm, zer_hbm, out_hbm,
              src_vm, dst_vm, buf, z_sh, acc_sh):
    c = lax.axis_index("c")
    s = lax.axis_index("s")
    wid = c * NS + s
    pltpu.sync_copy(src_hbm.at[wid], src_vm)
    pltpu.sync_copy(dst_hbm.at[wid], dst_vm)

    @pl.when(s == 0)
    def _():
        pltpu.sync_copy(z_hbm, z_sh)

    pltpu.sync_copy(zer_hbm, acc_sh.at[pl.ds(s * RPT, RPT)])
    plsc.subcore_barrier()

    def body(j, carry):
        pltpu.sync_copy(z_sh.at[src_vm.at[j]], buf)
        pltpu.sync_copy(buf, acc_sh.at[dst_vm.at[j]], add=True)
        return carry

    lax.fori_loop(0, BPW, body, 0)
    plsc.subcore_barrier()
    pltpu.sync_copy(acc_sh.at[pl.ds(s * RPT, RPT)],
                    out_hbm.at[pl.ds(c * NP + s * RPT, RPT)])


def _dis(hist_ref):
    h = hist_ref[...]
    deg = h[0, :N] + h[1, :N] + 1.0
    return lax.rsqrt(deg)


def _tc_y1_body(hist_ref, x_ref, w1_ref, y_ref):
    dis = _dis(hist_ref)
    xw = jnp.dot(x_ref[...], w1_ref[...], preferred_element_type=jnp.float32)
    y_ref[:, :HID] = xw * dis[:, None]
    y_ref[:, HID:] = jnp.zeros((N, HID), jnp.float32)


_tc_y1 = pl.pallas_call(
    _tc_y1_body,
    out_shape=jax.ShapeDtypeStruct((N, 2 * HID), jnp.float32),
)


def _tc_hz_body(hist_ref, s1_ref, y_ref, b1_ref, w2_ref, z_ref):
    dis = _dis(hist_ref)
    s1 = s1_ref[0, :N, :HID] + s1_ref[1, :N, :HID] + y_ref[:, :HID]
    hdn = jax.nn.relu(s1 * dis[:, None] + b1_ref[...])
    z = jnp.sum(hdn * w2_ref[...][:, 0][None, :], axis=1) * dis
    z_ref[...] = z[:, None]


_tc_hz = pl.pallas_call(
    _tc_hz_body,
    out_shape=jax.ShapeDtypeStruct((N, 1), jnp.float32),
)


def _tc_out_body(hist_ref, s2_ref, z_ref, b2_ref, o_ref):
    dis = _dis(hist_ref)
    s2 = s2_ref[0, :N] + s2_ref[1, :N] + z_ref[...][:, 0]
    o_ref[...] = (dis * s2 + b2_ref[...])[:, None]


_tc_out = pl.pallas_call(
    _tc_out_body,
    out_shape=jax.ShapeDtypeStruct((N, 1), jnp.float32),
)


def kernel(x, edge_index, W1, b1, W2, b2):
    src = edge_index[0].astype(jnp.int32)
    dst = edge_index[1].astype(jnp.int32)
    pad = E_PAD - E
    srcp = jnp.concatenate([src, jnp.zeros((pad,), jnp.int32)]).reshape(NW, BPW, K)
    dstp = jnp.concatenate([dst, jnp.full((pad,), N, jnp.int32)]).reshape(NW, BPW, K)
    na = NS * BPA * K
    srca = src[:na].reshape(NS, BPA, K)
    dsta = dst[:na].reshape(NS, BPA, K)
    srcb = jnp.concatenate([src[na:], jnp.zeros((pad,), jnp.int32)]
                           ).reshape(NS, BPB, K)
    dstb = jnp.concatenate([dst[na:], jnp.full((pad,), N, jnp.int32)]
                           ).reshape(NS, BPB, K)
    ones_k = jnp.ones((K,), jnp.float32)
    zer_r = jnp.zeros((RPT,), jnp.float32)
    zer_kh = jnp.zeros((RPT, 2 * HID), jnp.float32)

    hist = _sc_hist(dstp, ones_k, zer_r).reshape(NC, NP)
    y1 = _tc_y1(hist, x, W1)
    s1 = _sc_edge1(y1, srca, dsta, srcb, dstb, zer_kh).reshape(NC, NP, 2 * HID)
    z = _tc_hz(hist, s1, y1, b1, W2)
    s2 = _sc_edge2(z.reshape(N), srcp, dstp, zer_r).reshape(NC, NP)
    return _tc_out(hist, s2, z, b2)


# speedup vs baseline: 1.0097x; 1.0097x over previous
import functools

import jax
import jax.numpy as jnp
from jax import lax
from jax.experimental import pallas as pl
from jax.experimental.pallas import tpu as pltpu
from jax.experimental.pallas import tpu_sc as plsc

N = 10000
E = 320000
IN_DIM = 128
HID = 64
NC = 2
NS = 16
NW = NC * NS
K = 128
NBUF = 2
BPW = 80
EPW = BPW * K
E_PAD = NW * EPW
BPA = 80
BPB = 80
BCH = 40
CFAST = 1
NP = 10240
RPT = NP // NS

_MESH = plsc.VectorSubcoreMesh(core_axis_name="c", subcore_axis_name="s")


@functools.partial(
    pl.kernel,
    mesh=_MESH,
    out_type=jax.ShapeDtypeStruct((NC * NP,), jnp.float32),
    scratch_types=[
        pltpu.VMEM((BPW, K), jnp.int32),
        pltpu.VMEM((K,), jnp.float32),
        pltpu.VMEM_SHARED((NP,), jnp.float32),
    ],
)
def _sc_hist(dst_hbm, ones_hbm, zer_hbm, out_hbm, dst_vm, ones_vm, hist_sh):
    c = lax.axis_index("c")
    s = lax.axis_index("s")
    wid = c * NS + s
    pltpu.sync_copy(dst_hbm.at[wid], dst_vm)
    pltpu.sync_copy(ones_hbm, ones_vm)
    pltpu.sync_copy(zer_hbm, hist_sh.at[pl.ds(s * RPT, RPT)])
    plsc.subcore_barrier()

    def body(j, carry):
        pltpu.sync_copy(ones_vm, hist_sh.at[dst_vm.at[j]], add=True)
        return carry

    lax.fori_loop(0, BPW, body, 0)
    plsc.subcore_barrier()
    pltpu.sync_copy(hist_sh.at[pl.ds(s * RPT, RPT)],
                    out_hbm.at[pl.ds(c * NP + s * RPT, RPT)])


@functools.partial(
    pl.kernel,
    mesh=_MESH,
    out_type=jax.ShapeDtypeStruct((NC * NP, 2 * HID), jnp.float32),
    scratch_types=[
        pltpu.VMEM((BPA, K), jnp.int32),
        pltpu.VMEM((BPA, K), jnp.int32),
        pltpu.VMEM((K, 2 * HID), jnp.float32),
        pltpu.VMEM_SHARED((NP, 2 * HID), jnp.float32),
    ],
)
def _sc_edge1(y_hbm, srca_hbm, dsta_hbm, srcb_hbm, dstb_hbm, zer_hbm, out_hbm,
              src_vm, dst_vm, buf, acc_sh):
    c = lax.axis_index("c")
    s = lax.axis_index("s")
    pltpu.sync_copy(zer_hbm, acc_sh.at[pl.ds(s * RPT, RPT)])
    plsc.subcore_barrier()

    def run(src_hbm, dst_hbm, bpw):
        pltpu.sync_copy(src_hbm.at[s], src_vm.at[pl.ds(0, bpw)])
        pltpu.sync_copy(dst_hbm.at[s], dst_vm.at[pl.ds(0, bpw)])

        def body(j, carry):
            pltpu.sync_copy(y_hbm.at[src_vm.at[j]], buf)
            pltpu.sync_copy(buf, acc_sh.at[dst_vm.at[j]], add=True)
            return carry

        lax.fori_loop(0, bpw, body, 0)

    @pl.when(c == CFAST)
    def _():
        run(srca_hbm, dsta_hbm, BPA)

    @pl.when(c != CFAST)
    def _():
        run(srcb_hbm, dstb_hbm, BPB)

    plsc.subcore_barrier()
    pltpu.sync_copy(acc_sh.at[pl.ds(s * RPT, RPT)],
                    out_hbm.at[pl.ds(c * NP + s * RPT, RPT)])


@functools.partial(
    pl.kernel,
    mesh=_MESH,
    out_type=jax.ShapeDtypeStruct((NC * NP,), jnp.float32),
    scratch_types=[
        pltpu.VMEM((BPW, K), jnp.int32),
        pltpu.VMEM((BPW, K), jnp.int32),
        pltpu.VMEM((K,), jnp.float32),
        pltpu.VMEM_SHARED((N,), jnp.float32),
        pltpu.VMEM_SHARED((NP,), jnp.float32),
    ],
)
def _sc_edge2(z_hbm, src_hbm, dst_hbm, zer_hbm, out_hbm,
              src_vm, dst_vm, buf, z_sh, acc_sh):
    c = lax.axis_index("c")
    s = lax.axis_index("s")
    wid = c * NS + s
    pltpu.sync_copy(src_hbm.at[wid], src_vm)
    pltpu.sync_copy(dst_hbm.at[wid], dst_vm)

    @pl.when(s == 0)
    def _():
        pltpu.sync_copy(z_hbm, z_sh)

    pltpu.sync_copy(zer_hbm, acc_sh.at[pl.ds(s * RPT, RPT)])
    plsc.subcore_barrier()

    def body(j, carry):
        pltpu.sync_copy(z_sh.at[src_vm.at[j]], buf)
        pltpu.sync_copy(buf, acc_sh.at[dst_vm.at[j]], add=True)
        return carry

    lax.fori_loop(0, BPW, body, 0)
    plsc.subcore_barrier()
    pltpu.sync_copy(acc_sh.at[pl.ds(s * RPT, RPT)],
                    out_hbm.at[pl.ds(c * NP + s * RPT, RPT)])


def _dis(hist_ref):
    h = hist_ref[...]
    deg = h[0, :N] + h[1, :N] + 1.0
    return lax.rsqrt(deg)


def _tc_y1_body(hist_ref, x_ref, w1_ref, y_ref):
    dis = _dis(hist_ref)
    xw = jnp.dot(x_ref[...], w1_ref[...], preferred_element_type=jnp.float32)
    y_ref[:, :HID] = xw * dis[:, None]
    y_ref[:, HID:] = jnp.zeros((N, HID), jnp.float32)


_tc_y1 = pl.pallas_call(
    _tc_y1_body,
    out_shape=jax.ShapeDtypeStruct((N, 2 * HID), jnp.float32),
)


def _tc_hz_body(hist_ref, s1_ref, y_ref, b1_ref, w2_ref, z_ref):
    dis = _dis(hist_ref)
    s1 = s1_ref[0, :N, :HID] + s1_ref[1, :N, :HID] + y_ref[:, :HID]
    hdn = jax.nn.relu(s1 * dis[:, None] + b1_ref[...])
    z = jnp.sum(hdn * w2_ref[...][:, 0][None, :], axis=1) * dis
    z_ref[...] = z[:, None]


_tc_hz = pl.pallas_call(
    _tc_hz_body,
    out_shape=jax.ShapeDtypeStruct((N, 1), jnp.float32),
)


def _tc_out_body(hist_ref, s2_ref, z_ref, b2_ref, o_ref):
    dis = _dis(hist_ref)
    s2 = s2_ref[0, :N] + s2_ref[1, :N] + z_ref[...][:, 0]
    o_ref[...] = (dis * s2 + b2_ref[...])[:, None]


_tc_out = pl.pallas_call(
    _tc_out_body,
    out_shape=jax.ShapeDtypeStruct((N, 1), jnp.float32),
)


def kernel(x, edge_index, W1, b1, W2, b2):
    src = edge_index[0].astype(jnp.int32)
    dst = edge_index[1].astype(jnp.int32)
    pad = E_PAD - E
    pad_dst = N + jnp.arange(pad, dtype=jnp.int32) % (NP - N)
    srcp = jnp.concatenate([src, jnp.zeros((pad,), jnp.int32)]).reshape(NW, BPW, K)
    dstp = jnp.concatenate([dst, pad_dst]).reshape(NW, BPW, K)
    na = NS * BPA * K
    srca = src[:na].reshape(NS, BPA, K)
    dsta = dst[:na].reshape(NS, BPA, K)
    srcb = jnp.concatenate([src[na:], jnp.zeros((pad,), jnp.int32)]
                           ).reshape(NS, BPB, K)
    dstb = jnp.concatenate([dst[na:], pad_dst]).reshape(NS, BPB, K)
    ones_k = jnp.ones((K,), jnp.float32)
    zer_r = jnp.zeros((RPT,), jnp.float32)
    zer_kh = jnp.zeros((RPT, 2 * HID), jnp.float32)

    hist = _sc_hist(dstp, ones_k, zer_r).reshape(NC, NP)
    y1 = _tc_y1(hist, x, W1)
    s1 = _sc_edge1(y1, srca, dsta, srcb, dstb, zer_kh).reshape(NC, NP, 2 * HID)
    z = _tc_hz(hist, s1, y1, b1, W2)
    s2 = _sc_edge2(z.reshape(N), srcp, dstp, zer_r).reshape(NC, NP)
    return _tc_out(hist, s2, z, b2)


# speedup vs baseline: 2.2160x; 2.1948x over previous
import functools

import jax
import jax.numpy as jnp
from jax import lax
from jax.experimental import pallas as pl
from jax.experimental.pallas import tpu as pltpu
from jax.experimental.pallas import tpu_sc as plsc

N = 10000
E = 320000
IN_DIM = 128
HID = 64
NC = 2
NS = 16
NW = NC * NS
K = 128
NBUF = 2
BPW = 80
EPW = BPW * K
E_PAD = NW * EPW
BPA = 80
BPB = 80
BCH = 40
CFAST = 1
NP = 10240
RPT = NP // NS

_MESH = plsc.VectorSubcoreMesh(core_axis_name="c", subcore_axis_name="s")


@functools.partial(
    pl.kernel,
    mesh=_MESH,
    out_type=jax.ShapeDtypeStruct((NC * NP,), jnp.float32),
    scratch_types=[
        pltpu.VMEM((BPW, K), jnp.int32),
        pltpu.VMEM((K,), jnp.float32),
        pltpu.VMEM_SHARED((NP,), jnp.float32),
    ],
)
def _sc_hist(dst_hbm, ones_hbm, zer_hbm, out_hbm, dst_vm, ones_vm, hist_sh):
    c = lax.axis_index("c")
    s = lax.axis_index("s")
    wid = c * NS + s
    pltpu.sync_copy(dst_hbm.at[wid], dst_vm)
    pltpu.sync_copy(ones_hbm, ones_vm)
    pltpu.sync_copy(zer_hbm, hist_sh.at[pl.ds(s * RPT, RPT)])
    plsc.subcore_barrier()

    def body(j, carry):
        pltpu.sync_copy(ones_vm, hist_sh.at[dst_vm.at[j]], add=True)
        return carry

    lax.fori_loop(0, BPW, body, 0)
    plsc.subcore_barrier()
    pltpu.sync_copy(hist_sh.at[pl.ds(s * RPT, RPT)],
                    out_hbm.at[pl.ds(c * NP + s * RPT, RPT)])


@functools.partial(
    pl.kernel,
    mesh=_MESH,
    out_type=jax.ShapeDtypeStruct((NC * NP, 2 * HID), jnp.float32),
    scratch_types=[
        pltpu.VMEM((BPW, K), jnp.int32),
        pltpu.VMEM((BPW, K), jnp.int32),
        pltpu.VMEM((K, 2 * HID), jnp.float32),
        pltpu.VMEM_SHARED((NP, 2 * HID), jnp.float32),
    ],
)
def _sc_edge1(y_hbm, src_hbm, dst_hbm, zer_hbm, out_hbm,
              src_vm, dst_vm, buf, acc_sh):
    c = lax.axis_index("c")
    s = lax.axis_index("s")
    wid = c * NS + s
    pltpu.sync_copy(zer_hbm, acc_sh.at[pl.ds(s * RPT, RPT)])
    pltpu.sync_copy(src_hbm.at[wid], src_vm)
    pltpu.sync_copy(dst_hbm.at[wid], dst_vm)
    plsc.subcore_barrier()

    def body(j, carry):
        pltpu.sync_copy(y_hbm.at[src_vm.at[j]], buf)
        pltpu.sync_copy(buf, acc_sh.at[dst_vm.at[j]], add=True)
        return carry

    lax.fori_loop(0, BPW, body, 0)
    plsc.subcore_barrier()
    pltpu.sync_copy(acc_sh.at[pl.ds(s * RPT, RPT)],
                    out_hbm.at[pl.ds(c * NP + s * RPT, RPT)])


@functools.partial(
    pl.kernel,
    mesh=_MESH,
    out_type=jax.ShapeDtypeStruct((NC * NP,), jnp.float32),
    scratch_types=[
        pltpu.VMEM((BPW, K), jnp.int32),
        pltpu.VMEM((BPW, K), jnp.int32),
        pltpu.VMEM((K,), jnp.float32),
        pltpu.VMEM_SHARED((N,), jnp.float32),
        pltpu.VMEM_SHARED((NP,), jnp.float32),
    ],
)
def _sc_edge2(z_hbm, src_hbm, dst_hbm, zer_hbm, out_hbm,
              src_vm, dst_vm, buf, z_sh, acc_sh):
    c = lax.axis_index("c")
    s = lax.axis_index("s")
    wid = c * NS + s
    pltpu.sync_copy(src_hbm.at[wid], src_vm)
    pltpu.sync_copy(dst_hbm.at[wid], dst_vm)

    @pl.when(s == 0)
    def _():
        pltpu.sync_copy(z_hbm, z_sh)

    pltpu.sync_copy(zer_hbm, acc_sh.at[pl.ds(s * RPT, RPT)])
    plsc.subcore_barrier()

    def body(j, carry):
        pltpu.sync_copy(z_sh.at[src_vm.at[j]], buf)
        pltpu.sync_copy(buf, acc_sh.at[dst_vm.at[j]], add=True)
        return carry

    lax.fori_loop(0, BPW, body, 0)
    plsc.subcore_barrier()
    pltpu.sync_copy(acc_sh.at[pl.ds(s * RPT, RPT)],
                    out_hbm.at[pl.ds(c * NP + s * RPT, RPT)])


def _dis(hist_ref):
    h = hist_ref[...]
    deg = h[0, :N] + h[1, :N] + 1.0
    return lax.rsqrt(deg)


def _tc_y1_body(hist_ref, x_ref, w1_ref, y_ref):
    dis = _dis(hist_ref)
    xw = jnp.dot(x_ref[...], w1_ref[...], preferred_element_type=jnp.float32)
    y_ref[:, :HID] = xw * dis[:, None]
    y_ref[:, HID:] = jnp.zeros((N, HID), jnp.float32)


_tc_y1 = pl.pallas_call(
    _tc_y1_body,
    out_shape=jax.ShapeDtypeStruct((N, 2 * HID), jnp.float32),
)


def _tc_hz_body(hist_ref, s1_ref, y_ref, b1_ref, w2_ref, z_ref):
    dis = _dis(hist_ref)
    s1 = s1_ref[0, :N, :HID] + s1_ref[1, :N, :HID] + y_ref[:, :HID]
    hdn = jax.nn.relu(s1 * dis[:, None] + b1_ref[...])
    z = jnp.sum(hdn * w2_ref[...][:, 0][None, :], axis=1) * dis
    z_ref[...] = z[:, None]


_tc_hz = pl.pallas_call(
    _tc_hz_body,
    out_shape=jax.ShapeDtypeStruct((N, 1), jnp.float32),
)


def _tc_out_body(hist_ref, s2_ref, z_ref, b2_ref, o_ref):
    dis = _dis(hist_ref)
    s2 = s2_ref[0, :N] + s2_ref[1, :N] + z_ref[...][:, 0]
    o_ref[...] = (dis * s2 + b2_ref[...])[:, None]


_tc_out = pl.pallas_call(
    _tc_out_body,
    out_shape=jax.ShapeDtypeStruct((N, 1), jnp.float32),
)


def kernel(x, edge_index, W1, b1, W2, b2):
    src = edge_index[0].astype(jnp.int32)
    dst = edge_index[1].astype(jnp.int32)
    pad = E_PAD - E
    pad_dst = N + jnp.arange(pad, dtype=jnp.int32) % (NP - N)
    pad_src = jnp.arange(pad, dtype=jnp.int32) % N
    srcp = jnp.concatenate([src, pad_src]).reshape(NW, BPW, K)
    dstp = jnp.concatenate([dst, pad_dst]).reshape(NW, BPW, K)
    ones_k = jnp.ones((K,), jnp.float32)
    zer_r = jnp.zeros((RPT,), jnp.float32)
    zer_kh = jnp.zeros((RPT, 2 * HID), jnp.float32)

    hist = _sc_hist(dstp, ones_k, zer_r).reshape(NC, NP)
    y1 = _tc_y1(hist, x, W1)
    s1 = _sc_edge1(y1, srcp, dstp, zer_kh).reshape(NC, NP, 2 * HID)
    z = _tc_hz(hist, s1, y1, b1, W2)
    s2 = _sc_edge2(z.reshape(N), srcp, dstp, zer_r).reshape(NC, NP)
    return _tc_out(hist, s2, z, b2)
